# Initial kernel scaffold; baseline (speedup 1.0000x reference)
import jax
import jax.numpy as jnp
from jax.experimental import pallas as pl


def kernel(h, edge_index, W1, b1, Wr1, br1, W2, b2, Wr2, br2):
    raise NotImplementedError("write your pallas kernel here")


# speedup vs baseline: 3.6684x; 3.6684x over previous
import functools

import jax
import jax.numpy as jnp
from jax import lax
from jax.experimental import pallas as pl
from jax.experimental.pallas import tpu as pltpu
from jax.experimental.pallas import tpu_sc as plsc

N = 10000
E = 320000
D = 128
H = 128

NC = 2
NS = 16
NW = NC * NS

CHUNK = 128
NCHUNK = -(-E // (NW * CHUNK))
EPW = NCHUNK * CHUNK
E_PAD = EPW * NW

SZ = 640
N_ACC = SZ * NS

_f32 = jnp.float32


def _dense_first(h, W, Wr, br):
    def body(h_ref, W_ref, Wr_ref, br_ref, m_ref, res_ref):
        hb = h_ref[...]
        m_ref[...] = jnp.dot(hb, W_ref[...], preferred_element_type=_f32)
        res_ref[...] = jnp.maximum(
            jnp.dot(hb, Wr_ref[...], preferred_element_type=_f32) + br_ref[...], 0.0)

    BN = 1000
    return pl.pallas_call(
        body,
        grid=(N // BN,),
        in_specs=[
            pl.BlockSpec((BN, D), lambda i: (i, 0)),
            pl.BlockSpec((D, H), lambda i: (0, 0)),
            pl.BlockSpec((D, H), lambda i: (0, 0)),
            pl.BlockSpec((1, H), lambda i: (0, 0)),
        ],
        out_specs=[
            pl.BlockSpec((BN, H), lambda i: (i, 0)),
            pl.BlockSpec((BN, H), lambda i: (i, 0)),
        ],
        out_shape=[jax.ShapeDtypeStruct((N, H), _f32)] * 2,
    )(h, W, Wr, br.reshape(1, H))


def _dense_mid(p, b, res, W, Wr, br):
    def body(p_ref, b_ref, res_ref, W_ref, Wr_ref, br_ref, m_ref, res2_ref):
        h1 = jnp.maximum(p_ref[0] + p_ref[1] + b_ref[...], 0.0) + res_ref[...]
        m_ref[...] = jnp.dot(h1, W_ref[...], preferred_element_type=_f32)
        res2_ref[...] = jnp.maximum(
            jnp.dot(h1, Wr_ref[...], preferred_element_type=_f32) + br_ref[...], 0.0)

    BN = 1000
    return pl.pallas_call(
        body,
        grid=(N // BN,),
        in_specs=[
            pl.BlockSpec((2, BN, H), lambda i: (0, i, 0)),
            pl.BlockSpec((1, H), lambda i: (0, 0)),
            pl.BlockSpec((BN, H), lambda i: (i, 0)),
            pl.BlockSpec((H, H), lambda i: (0, 0)),
            pl.BlockSpec((H, H), lambda i: (0, 0)),
            pl.BlockSpec((1, H), lambda i: (0, 0)),
        ],
        out_specs=[
            pl.BlockSpec((BN, H), lambda i: (i, 0)),
            pl.BlockSpec((BN, H), lambda i: (i, 0)),
        ],
        out_shape=[jax.ShapeDtypeStruct((N, H), _f32)] * 2,
    )(p, b.reshape(1, H), res, W, Wr, br.reshape(1, H))


def _combine_last(p, b, res):
    def body(p_ref, b_ref, res_ref, o_ref):
        o_ref[...] = (
            jnp.maximum(p_ref[0] + p_ref[1] + b_ref[...], 0.0) + res_ref[...])

    BN = 1000
    return pl.pallas_call(
        body,
        grid=(N // BN,),
        in_specs=[
            pl.BlockSpec((2, BN, H), lambda i: (0, i, 0)),
            pl.BlockSpec((1, H), lambda i: (0, 0)),
            pl.BlockSpec((BN, H), lambda i: (i, 0)),
        ],
        out_specs=pl.BlockSpec((BN, H), lambda i: (i, 0)),
        out_shape=jax.ShapeDtypeStruct((N, H), _f32),
    )(p, b.reshape(1, H), res)


def _sc_segment_sum(m, srcp, dstp, zeros):
    mesh = plsc.VectorSubcoreMesh(
        core_axis_name="c", subcore_axis_name="s",
        num_cores=NC, num_subcores=NS)

    @functools.partial(
        pl.kernel,
        mesh=mesh,
        out_type=jax.ShapeDtypeStruct((NC, N_ACC, H), _f32),
        scratch_types=[
            pltpu.VMEM((CHUNK,), jnp.int32),
            pltpu.VMEM((CHUNK,), jnp.int32),
            pltpu.VMEM((CHUNK, H), _f32),
            pltpu.VMEM_SHARED((N_ACC, H), _f32),
            pltpu.SemaphoreType.DMA,
        ],
    )
    def sc_kernel(m_hbm, src_hbm, dst_hbm, z_hbm, out_hbm,
                  sidx_v, didx_v, rows_v, acc_sh, sem):
        c = lax.axis_index("c")
        s = lax.axis_index("s")
        wid = s * NC + c
        base = wid * EPW

        pltpu.sync_copy(z_hbm, acc_sh.at[pl.ds(s * SZ, SZ)])
        plsc.subcore_barrier()

        @pl.loop(0, NCHUNK)
        def _chunk(t):
            off = base + t * CHUNK
            pltpu.sync_copy(src_hbm.at[pl.ds(off, CHUNK)], sidx_v)
            pltpu.sync_copy(dst_hbm.at[pl.ds(off, CHUNK)], didx_v)
            pltpu.async_copy(m_hbm.at[sidx_v], rows_v, sem).wait()
            pltpu.sync_copy(rows_v, acc_sh.at[didx_v], add=True)

        plsc.subcore_barrier()
        pltpu.sync_copy(acc_sh.at[pl.ds(s * SZ, SZ)],
                        out_hbm.at[c, pl.ds(s * SZ, SZ)])

    return sc_kernel(m, srcp, dstp, zeros)


def kernel(h, edge_index, W1, b1, Wr1, br1, W2, b2, Wr2, br2):
    src = edge_index[0].astype(jnp.int32)
    dst = edge_index[1].astype(jnp.int32)
    pad = E_PAD - E
    srcp = jnp.concatenate([src, jnp.zeros((pad,), jnp.int32)])
    dstp = jnp.concatenate([dst, jnp.full((pad,), N, jnp.int32)])
    zeros = jnp.zeros((SZ, H), _f32)

    m1, res1 = _dense_first(h, W1, Wr1, br1)
    p1 = _sc_segment_sum(m1, srcp, dstp, zeros)
    m2, res2 = _dense_mid(p1, b1, res1, W2, Wr2, br2)
    p2 = _sc_segment_sum(m2, srcp, dstp, zeros)
    return _combine_last(p2, b2, res2)
